# Initial kernel scaffold; baseline (speedup 1.0000x reference)
import functools

import jax
import jax.numpy as jnp
from jax.experimental import pallas as pl
from jax.experimental.pallas import tpu as pltpu

_N = 10000
_M = 2500
_R2 = 0.2 * 0.2
_K = 64
_NPAD = 10240
_ROWS = 80


def _fps_body(px_ref, py_ref, pz_ref, mind0_ref, out_ref, mind_s):
    mind_s[...] = mind0_ref[...]
    flat = (jax.lax.broadcasted_iota(jnp.int32, (_ROWS, 128), 0) * 128
            + jax.lax.broadcasted_iota(jnp.int32, (_ROWS, 128), 1))
    out_ref[0] = 0

    def body(i, _):
        px = px_ref[...]
        py = py_ref[...]
        pz = pz_ref[...]
        mind = mind_s[...]
        m = jnp.max(mind)
        eq = mind == m
        nxt = jnp.min(jnp.where(eq, flat, jnp.int32(2**30)))
        eq2 = flat == nxt
        qx = jnp.sum(jnp.where(eq2, px, 0.0))
        qy = jnp.sum(jnp.where(eq2, py, 0.0))
        qz = jnp.sum(jnp.where(eq2, pz, 0.0))
        dx = px - qx
        dy = py - qy
        dz = pz - qz
        d = dx * dx + dy * dy + dz * dz
        mind_s[...] = jnp.minimum(mind, d)
        out_ref[i] = nxt
        return 0

    jax.lax.fori_loop(1, _M, body, 0)


def _fps(pos):
    pad = _NPAD - _N
    px = jnp.pad(pos[:, 0], (0, pad)).reshape(_ROWS, 128)
    py = jnp.pad(pos[:, 1], (0, pad)).reshape(_ROWS, 128)
    pz = jnp.pad(pos[:, 2], (0, pad)).reshape(_ROWS, 128)
    d0 = jnp.sum((pos - pos[0]) ** 2, axis=1)
    mind0 = jnp.pad(d0, (0, pad), constant_values=-1.0).reshape(_ROWS, 128)
    return pl.pallas_call(
        _fps_body,
        out_shape=jax.ShapeDtypeStruct((_M,), jnp.int32),
        in_specs=[pl.BlockSpec(memory_space=pltpu.VMEM)] * 4,
        out_specs=pl.BlockSpec(memory_space=pltpu.SMEM),
        scratch_shapes=[pltpu.VMEM((_ROWS, 128), jnp.float32)],
    )(px, py, pz, mind0)


_CB = 50


def _conv_body(xj_ref, rel_ref, w1x_ref, w1p_ref, b1_ref, w2_ref, b2_ref, out_ref):
    h = jnp.dot(xj_ref[...], w1x_ref[...], preferred_element_type=jnp.float32)
    h = h + jnp.dot(rel_ref[...], w1p_ref[...], preferred_element_type=jnp.float32)
    h = jnp.maximum(h + b1_ref[...], 0.0)
    h = jnp.dot(h, w2_ref[...], preferred_element_type=jnp.float32) + b2_ref[...]
    h = jnp.maximum(h, 0.0)
    out_ref[...] = jnp.max(h.reshape(_CB, _K, 128), axis=1)


def _conv(xj, rel, W1, b1, W2, b2):
    w1x = W1[:128]
    w1p = jnp.pad(W1[128:], ((0, 5), (0, 0)))
    grid = _M // _CB
    return pl.pallas_call(
        _conv_body,
        grid=(grid,),
        in_specs=[
            pl.BlockSpec((_CB * _K, 128), lambda i: (i, 0)),
            pl.BlockSpec((_CB * _K, 8), lambda i: (i, 0)),
            pl.BlockSpec((128, 128), lambda i: (0, 0)),
            pl.BlockSpec((8, 128), lambda i: (0, 0)),
            pl.BlockSpec((1, 128), lambda i: (0, 0)),
            pl.BlockSpec((128, 128), lambda i: (0, 0)),
            pl.BlockSpec((1, 128), lambda i: (0, 0)),
        ],
        out_specs=pl.BlockSpec((_CB, 128), lambda i: (i, 0)),
        out_shape=jax.ShapeDtypeStruct((_M, 128), jnp.float32),
    )(xj, rel, w1x, w1p, b1.reshape(1, 128), W2, b2.reshape(1, 128))


def kernel(x, pos, batch, W1, b1, W2, b2):
    idx = _fps(pos)
    pos_q = pos[idx]

    sq = jnp.sum((pos_q[:, None, :] - pos[None, :, :]) ** 2, axis=-1)
    neg = jnp.where(sq < _R2, -sq, -jnp.inf)
    vals, nbr = jax.lax.top_k(neg, _K)
    valid = vals > -jnp.inf
    nbr = jnp.where(valid, nbr, nbr[:, :1])

    xj = x[nbr].reshape(_M * _K, 128)
    rel = (pos[nbr] - pos_q[:, None, :]).reshape(_M * _K, 3)
    rel = jnp.pad(rel, ((0, 0), (0, 5)))
    out = _conv(xj, rel, W1, b1, W2, b2)
    return (out, pos_q, batch[idx])


# speedup vs baseline: 1.2240x; 1.2240x over previous
import functools

import jax
import jax.numpy as jnp
from jax.experimental import pallas as pl
from jax.experimental.pallas import tpu as pltpu

_N = 10000
_M = 2500
_R2 = 0.2 * 0.2
_K = 64
_NPAD = 10240
_ROWS = 80


def _fps_body(px_ref, py_ref, pz_ref, mind0_ref, out_ref, mind_s):
    mind_s[...] = mind0_ref[...]
    flat = (jax.lax.broadcasted_iota(jnp.int32, (_ROWS, 128), 0) * 128
            + jax.lax.broadcasted_iota(jnp.int32, (_ROWS, 128), 1))
    out_ref[0] = 0

    def body(i, _):
        px = px_ref[...]
        py = py_ref[...]
        pz = pz_ref[...]
        mind = mind_s[...]
        m = jnp.max(mind)
        eq = mind == m
        nxt = jnp.min(jnp.where(eq, flat, jnp.int32(2**30)))
        eq2 = flat == nxt
        qx = jnp.sum(jnp.where(eq2, px, 0.0))
        qy = jnp.sum(jnp.where(eq2, py, 0.0))
        qz = jnp.sum(jnp.where(eq2, pz, 0.0))
        dx = px - qx
        dy = py - qy
        dz = pz - qz
        d = dx * dx + dy * dy + dz * dz
        mind_s[...] = jnp.minimum(mind, d)
        out_ref[i] = nxt
        return 0

    jax.lax.fori_loop(1, _M, body, 0)


def _fps(pos):
    pad = _NPAD - _N
    px = jnp.pad(pos[:, 0], (0, pad)).reshape(_ROWS, 128)
    py = jnp.pad(pos[:, 1], (0, pad)).reshape(_ROWS, 128)
    pz = jnp.pad(pos[:, 2], (0, pad)).reshape(_ROWS, 128)
    d0 = jnp.sum((pos - pos[0]) ** 2, axis=1)
    mind0 = jnp.pad(d0, (0, pad), constant_values=-1.0).reshape(_ROWS, 128)
    return pl.pallas_call(
        _fps_body,
        out_shape=jax.ShapeDtypeStruct((_M,), jnp.int32),
        in_specs=[pl.BlockSpec(memory_space=pltpu.VMEM)] * 4,
        out_specs=pl.BlockSpec(memory_space=pltpu.SMEM),
        scratch_shapes=[pltpu.VMEM((_ROWS, 128), jnp.float32)],
    )(px, py, pz, mind0)


_MPAD = 2560
_CB = 64


def _conv_body(xj_ref, rel_ref, w1x_ref, w1p_ref, b1_ref, w2_ref, b2_ref, out_ref):
    h = jnp.dot(xj_ref[...], w1x_ref[...], preferred_element_type=jnp.float32)
    h = h + jnp.dot(rel_ref[...], w1p_ref[...], preferred_element_type=jnp.float32)
    h = jnp.maximum(h + b1_ref[...], 0.0)
    h = jnp.dot(h, w2_ref[...], preferred_element_type=jnp.float32) + b2_ref[...]
    h = jnp.maximum(h, 0.0)
    out_ref[...] = jnp.max(h.reshape(_CB, _K, 128), axis=1)


def _conv(xj, rel, W1, b1, W2, b2):
    w1x = W1[:128]
    w1p = jnp.pad(W1[128:], ((0, 5), (0, 0)))
    grid = _MPAD // _CB
    return pl.pallas_call(
        _conv_body,
        grid=(grid,),
        in_specs=[
            pl.BlockSpec((_CB * _K, 128), lambda i: (i, 0)),
            pl.BlockSpec((_CB * _K, 8), lambda i: (i, 0)),
            pl.BlockSpec((128, 128), lambda i: (0, 0)),
            pl.BlockSpec((8, 128), lambda i: (0, 0)),
            pl.BlockSpec((1, 128), lambda i: (0, 0)),
            pl.BlockSpec((128, 128), lambda i: (0, 0)),
            pl.BlockSpec((1, 128), lambda i: (0, 0)),
        ],
        out_specs=pl.BlockSpec((_CB, 128), lambda i: (i, 0)),
        out_shape=jax.ShapeDtypeStruct((_MPAD, 128), jnp.float32),
    )(xj, rel, w1x, w1p, b1.reshape(1, 128), W2, b2.reshape(1, 128))


def kernel(x, pos, batch, W1, b1, W2, b2):
    idx = _fps(pos)
    pos_q = pos[idx]

    sq = jnp.sum((pos_q[:, None, :] - pos[None, :, :]) ** 2, axis=-1)
    neg = jnp.where(sq < _R2, -sq, -jnp.inf)
    vals, nbr = jax.lax.top_k(neg, _K)
    valid = vals > -jnp.inf
    nbr = jnp.where(valid, nbr, nbr[:, :1])

    xj = x[nbr].reshape(_M * _K, 128)
    xj = jnp.pad(xj, ((0, (_MPAD - _M) * _K), (0, 0)))
    rel = (pos[nbr] - pos_q[:, None, :]).reshape(_M * _K, 3)
    rel = jnp.pad(rel, ((0, (_MPAD - _M) * _K), (0, 5)))
    out = _conv(xj, rel, W1, b1, W2, b2)[:_M]
    return (out, pos_q, batch[idx])


# speedup vs baseline: 14.3892x; 11.7562x over previous
import functools

import jax
import jax.numpy as jnp
from jax import lax
from jax.experimental import pallas as pl
from jax.experimental.pallas import tpu as pltpu
from jax.experimental.pallas import tpu_sc as plsc

_N = 10000
_M = 2500
_R2 = 0.2 * 0.2
_K = 64
_NPAD = 10240
_ROWS = 80
_MPAD = 2560
_NW = 32
_RW = _MPAD // _NW
_CAP = 768
_BIGBITS = 0x7F700000
_R2BITS = __import__("struct").unpack("<i", __import__("struct").pack("<f", _R2))[0]


def _sc_mesh():
    return plsc.VectorSubcoreMesh(
        core_axis_name="c", subcore_axis_name="s", num_cores=2, num_subcores=16)


def _fps_body(px_ref, py_ref, pz_ref, mind0_ref, out_ref, mind_s):
    mind_s[...] = mind0_ref[...]
    flat = (lax.broadcasted_iota(jnp.int32, (_ROWS, 128), 0) * 128
            + lax.broadcasted_iota(jnp.int32, (_ROWS, 128), 1))
    out_ref[0] = 0

    def body(i, _):
        px = px_ref[...]
        py = py_ref[...]
        pz = pz_ref[...]
        mind = mind_s[...]
        m = jnp.max(mind)
        eq = mind == m
        nxt = jnp.min(jnp.where(eq, flat, jnp.int32(2**30)))
        eq2 = flat == nxt
        qx = jnp.sum(jnp.where(eq2, px, 0.0))
        qy = jnp.sum(jnp.where(eq2, py, 0.0))
        qz = jnp.sum(jnp.where(eq2, pz, 0.0))
        dx = px - qx
        dy = py - qy
        dz = pz - qz
        d = dx * dx + (dy * dy + dz * dz)
        mind_s[...] = jnp.minimum(mind, d)
        out_ref[i] = nxt
        return 0

    lax.fori_loop(1, _M, body, 0)


def _fps(pos):
    pad = _NPAD - _N
    px = jnp.pad(pos[:, 0], (0, pad)).reshape(_ROWS, 128)
    py = jnp.pad(pos[:, 1], (0, pad)).reshape(_ROWS, 128)
    pz = jnp.pad(pos[:, 2], (0, pad)).reshape(_ROWS, 128)
    d0 = jnp.sum((pos - pos[0]) ** 2, axis=1)
    mind0 = jnp.pad(d0, (0, pad), constant_values=-1.0).reshape(_ROWS, 128)
    return pl.pallas_call(
        _fps_body,
        out_shape=jax.ShapeDtypeStruct((_M,), jnp.int32),
        in_specs=[pl.BlockSpec(memory_space=pltpu.VMEM)] * 4,
        out_specs=pl.BlockSpec(memory_space=pltpu.SMEM),
        scratch_shapes=[pltpu.VMEM((_ROWS, 128), jnp.float32)],
    )(px, py, pz, mind0)


def _ballq_body(px_hbm, py_hbm, pz_hbm, qx_hbm, qy_hbm, qz_hbm, iq_hbm,
                nbr_hbm, px_v, py_v, pz_v, qx_v, qy_v, qz_v, iq_v,
                cand_v, bits_v, eq_v, sel_v, sem):
    wid = lax.axis_index("s") * 2 + lax.axis_index("c")
    rbase = wid * _RW
    pltpu.sync_copy(px_hbm.at[pl.ds(0, _NPAD)], px_v)
    pltpu.sync_copy(py_hbm.at[pl.ds(0, _NPAD)], py_v)
    pltpu.sync_copy(pz_hbm.at[pl.ds(0, _NPAD)], pz_v)
    pltpu.sync_copy(qx_hbm.at[pl.ds(rbase, _RW)], qx_v)
    pltpu.sync_copy(qy_hbm.at[pl.ds(rbase, _RW)], qy_v)
    pltpu.sync_copy(qz_hbm.at[pl.ds(rbase, _RW)], qz_v)
    pltpu.sync_copy(iq_hbm.at[pl.ds(rbase, _RW)], iq_v)

    iota = lax.iota(jnp.int32, 16)
    lane_base = iota * _CAP

    def group(g, _):
        qxg = qx_v[pl.ds(g * 16, 16)]
        qyg = qy_v[pl.ds(g * 16, 16)]
        qzg = qz_v[pl.ds(g * 16, 16)]

        def scan(c, off):
            lx = px_v[pl.ds(c * 16, 16)]
            ly = py_v[pl.ds(c * 16, 16)]
            lz = pz_v[pl.ds(c * 16, 16)]
            for l in range(16):
                sp = jnp.full((16,), l, jnp.int32)
                dx = jnp.take(lx, sp) - qxg
                dy = jnp.take(ly, sp) - qyg
                dz = jnp.take(lz, sp) - qzg
                d = dx * dx + (dy * dy + dz * dz)
                mm = d < _R2
                tgt = lane_base + jnp.minimum(off, _CAP - 1)
                plsc.store_scatter(cand_v, [tgt], c * 16 + l + jnp.zeros((16,), jnp.int32), mask=mm)
                off = off + mm.astype(jnp.int32)
            return off

        off = lax.fori_loop(0, _NPAD // 16, scan, jnp.zeros((16,), jnp.int32))

        for l in range(16):
            rloc = g * 16 + l
            cnt = jnp.minimum(off[l], _CAP - 1)
            k_take = jnp.minimum(cnt, _K)
            nc = (cnt + 15) // 16
            qxb = jnp.take(qxg, jnp.full((16,), l, jnp.int32))
            qyb = jnp.take(qyg, jnp.full((16,), l, jnp.int32))
            qzb = jnp.take(qzg, jnp.full((16,), l, jnp.int32))

            def rebuild(ch, _):
                ci = cand_v[pl.ds(l * _CAP + ch * 16, 16)]
                ci = jnp.clip(ci, 0, _NPAD - 1)
                gx = plsc.load_gather(px_v, [ci]) - qxb
                gy = plsc.load_gather(py_v, [ci]) - qyb
                gz = plsc.load_gather(pz_v, [ci]) - qzb
                d = gx * gx + (gy * gy + gz * gz)
                db = plsc.bitcast(d, jnp.int32)
                lane = ch * 16 + iota
                db = jnp.where(lane < cnt, db, _BIGBITS)
                bits_v[pl.ds(ch * 16, 16)] = db
                return 0

            lax.fori_loop(0, nc, rebuild, 0)

            def count_le(t):
                def cbody(ch, acc):
                    db = bits_v[pl.ds(ch * 16, 16)]
                    return acc + plsc.all_reduce_population_count(db <= t)

                acc = lax.fori_loop(0, nc, cbody, jnp.zeros((16,), jnp.int32))
                return acc[0]

            def bis(_, lohi):
                lo, hi = lohi
                mid = lo + (hi - lo) // 2
                c = count_le(mid)
                return jnp.where(c >= k_take, lo, mid), jnp.where(c >= k_take, mid, hi)

            lo, hi = lax.fori_loop(0, 31, bis, (jnp.int32(-1), jnp.int32(_R2BITS)))
            t_star = hi
            c_lt = count_le(t_star - 1)
            need_eq = k_take - c_lt

            self_sp = plsc.load_gather(iq_v, [jnp.full((16,), rloc, jnp.int32)])
            for ch4 in range(4):
                sel_v[pl.ds(rloc * _K + ch4 * 16, 16)] = self_sp

            def emit(ch, offs):
                o_lt, o_eq = offs
                db = bits_v[pl.ds(ch * 16, 16)]
                ci = cand_v[pl.ds(l * _CAP + ch * 16, 16)]
                m_lt = db < t_star
                m_eq = db == t_star
                plsc.store_compressed(sel_v.at[pl.ds(rloc * _K + o_lt, 16)], ci, mask=m_lt)
                plsc.store_compressed(eq_v.at[pl.ds(o_eq, 16)], ci, mask=m_eq)
                return (o_lt + plsc.all_reduce_population_count(m_lt)[0],
                        o_eq + plsc.all_reduce_population_count(m_eq)[0])

            lax.fori_loop(0, nc, emit, (jnp.int32(0), jnp.int32(0)))

            for ch4 in range(4):
                mpref = (ch4 * 16 + iota) < need_eq
                ev = eq_v[pl.ds(ch4 * 16, 16)]
                o3 = rloc * _K + jnp.minimum(c_lt + ch4 * 16, _K)
                plsc.store_compressed(sel_v.at[pl.ds(o3, 16)], ev, mask=mpref)
        return 0

    lax.fori_loop(0, _RW // 16, group, 0)
    pltpu.sync_copy(sel_v.at[pl.ds(0, _RW * _K)], nbr_hbm.at[pl.ds(rbase * _K, _RW * _K)])


def _ballq(px, py, pz, qx, qy, qz, iq):
    f = functools.partial(
        pl.kernel,
        out_type=jax.ShapeDtypeStruct((_MPAD * _K,), jnp.int32),
        mesh=_sc_mesh(),
        scratch_types=[
            pltpu.VMEM((_NPAD,), jnp.float32),
            pltpu.VMEM((_NPAD,), jnp.float32),
            pltpu.VMEM((_NPAD,), jnp.float32),
            pltpu.VMEM((_RW,), jnp.float32),
            pltpu.VMEM((_RW,), jnp.float32),
            pltpu.VMEM((_RW,), jnp.float32),
            pltpu.VMEM((_RW,), jnp.int32),
            pltpu.VMEM((16 * _CAP,), jnp.int32),
            pltpu.VMEM((_CAP + 16,), jnp.int32),
            pltpu.VMEM((_CAP + 16,), jnp.int32),
            pltpu.VMEM((_RW * _K + 16,), jnp.int32),
            pltpu.SemaphoreType.DMA,
        ],
        compiler_params=pltpu.CompilerParams(needs_layout_passes=False),
    )(_ballq_body)
    return f(px, py, pz, qx, qy, qz, iq)


_GC = 512


def _gather_body(x_hbm, nbr_hbm, px_hbm, py_hbm, pz_hbm, qx_hbm, qy_hbm, qz_hbm,
                 xg_hbm, rx_hbm, ry_hbm, rz_hbm,
                 px_v, py_v, pz_v, qx_v, qy_v, qz_v, idx_v, rows_v,
                 rx_v, ry_v, rz_v, sem):
    wid = lax.axis_index("s") * 2 + lax.axis_index("c")
    rbase = wid * _RW
    fbase = rbase * _K
    pltpu.sync_copy(px_hbm.at[pl.ds(0, _NPAD)], px_v)
    pltpu.sync_copy(py_hbm.at[pl.ds(0, _NPAD)], py_v)
    pltpu.sync_copy(pz_hbm.at[pl.ds(0, _NPAD)], pz_v)
    pltpu.sync_copy(qx_hbm.at[pl.ds(rbase, _RW)], qx_v)
    pltpu.sync_copy(qy_hbm.at[pl.ds(rbase, _RW)], qy_v)
    pltpu.sync_copy(qz_hbm.at[pl.ds(rbase, _RW)], qz_v)

    def chunk(ch, _):
        pltpu.sync_copy(nbr_hbm.at[pl.ds(fbase + ch * _GC, _GC)], idx_v)
        cp = pltpu.async_copy(x_hbm.at[idx_v], rows_v, sem)

        def sub(sc, _):
            ci = idx_v[pl.ds(sc * 16, 16)]
            rloc = ch * (_GC // _K) + sc // 4
            sp = jnp.full((16,), rloc, jnp.int32)
            gx = plsc.load_gather(px_v, [ci]) - plsc.load_gather(qx_v, [sp])
            gy = plsc.load_gather(py_v, [ci]) - plsc.load_gather(qy_v, [sp])
            gz = plsc.load_gather(pz_v, [ci]) - plsc.load_gather(qz_v, [sp])
            rx_v[pl.ds(sc * 16, 16)] = gx
            ry_v[pl.ds(sc * 16, 16)] = gy
            rz_v[pl.ds(sc * 16, 16)] = gz
            return 0

        lax.fori_loop(0, _GC // 16, sub, 0)
        pltpu.sync_copy(rx_v, rx_hbm.at[pl.ds(fbase + ch * _GC, _GC)])
        pltpu.sync_copy(ry_v, ry_hbm.at[pl.ds(fbase + ch * _GC, _GC)])
        pltpu.sync_copy(rz_v, rz_hbm.at[pl.ds(fbase + ch * _GC, _GC)])
        cp.wait()
        pltpu.sync_copy(rows_v, xg_hbm.at[pl.ds(fbase + ch * _GC, _GC)])
        return 0

    lax.fori_loop(0, _RW * _K // _GC, chunk, 0)


def _gather(x, nbr_flat, px, py, pz, qx, qy, qz):
    f = functools.partial(
        pl.kernel,
        out_type=(
            jax.ShapeDtypeStruct((_MPAD * _K, 128), jnp.float32),
            jax.ShapeDtypeStruct((_MPAD * _K,), jnp.float32),
            jax.ShapeDtypeStruct((_MPAD * _K,), jnp.float32),
            jax.ShapeDtypeStruct((_MPAD * _K,), jnp.float32),
        ),
        mesh=_sc_mesh(),
        scratch_types=[
            pltpu.VMEM((_NPAD,), jnp.float32),
            pltpu.VMEM((_NPAD,), jnp.float32),
            pltpu.VMEM((_NPAD,), jnp.float32),
            pltpu.VMEM((_RW,), jnp.float32),
            pltpu.VMEM((_RW,), jnp.float32),
            pltpu.VMEM((_RW,), jnp.float32),
            pltpu.VMEM((_GC,), jnp.int32),
            pltpu.VMEM((_GC, 128), jnp.float32),
            pltpu.VMEM((_GC,), jnp.float32),
            pltpu.VMEM((_GC,), jnp.float32),
            pltpu.VMEM((_GC,), jnp.float32),
            pltpu.SemaphoreType.DMA,
        ],
        compiler_params=pltpu.CompilerParams(needs_layout_passes=False),
    )(_gather_body)
    return f(x, nbr_flat, px, py, pz, qx, qy, qz)


_CB = 64


def _conv_body(xj_ref, rx_ref, ry_ref, rz_ref, w1x_ref, p0_ref, p1_ref, p2_ref,
               b1_ref, w2_ref, b2_ref, out_ref):
    h = jnp.dot(xj_ref[...], w1x_ref[...], preferred_element_type=jnp.float32)
    h = h + rx_ref[...] * p0_ref[...]
    h = h + ry_ref[...] * p1_ref[...]
    h = h + rz_ref[...] * p2_ref[...]
    h = jnp.maximum(h + b1_ref[...], 0.0)
    h = jnp.dot(h, w2_ref[...], preferred_element_type=jnp.float32) + b2_ref[...]
    h = jnp.maximum(h, 0.0)
    out_ref[...] = jnp.max(h.reshape(_CB, _K, 128), axis=1)


def _conv(xj, rx, ry, rz, W1, b1, W2, b2):
    w1x = W1[:128]
    grid = _MPAD // _CB
    full = lambda i: (0, 0)
    return pl.pallas_call(
        _conv_body,
        grid=(grid,),
        in_specs=[
            pl.BlockSpec((_CB * _K, 128), lambda i: (i, 0)),
            pl.BlockSpec((_CB * _K, 1), lambda i: (i, 0)),
            pl.BlockSpec((_CB * _K, 1), lambda i: (i, 0)),
            pl.BlockSpec((_CB * _K, 1), lambda i: (i, 0)),
            pl.BlockSpec((128, 128), full),
            pl.BlockSpec((1, 128), full),
            pl.BlockSpec((1, 128), full),
            pl.BlockSpec((1, 128), full),
            pl.BlockSpec((1, 128), full),
            pl.BlockSpec((128, 128), full),
            pl.BlockSpec((1, 128), full),
        ],
        out_specs=pl.BlockSpec((_CB, 128), lambda i: (i, 0)),
        out_shape=jax.ShapeDtypeStruct((_MPAD, 128), jnp.float32),
    )(xj, rx.reshape(-1, 1), ry.reshape(-1, 1), rz.reshape(-1, 1),
      w1x, W1[128].reshape(1, 128), W1[129].reshape(1, 128), W1[130].reshape(1, 128),
      b1.reshape(1, 128), W2, b2.reshape(1, 128))


def kernel(x, pos, batch, W1, b1, W2, b2):
    idx = _fps(pos)
    pos_q = pos[idx]

    pad = _NPAD - _N
    px = jnp.pad(pos[:, 0], (0, pad), constant_values=1e3)
    py = jnp.pad(pos[:, 1], (0, pad), constant_values=1e3)
    pz = jnp.pad(pos[:, 2], (0, pad), constant_values=1e3)
    qpad = _MPAD - _M
    qx = jnp.pad(pos_q[:, 0], (0, qpad), constant_values=2e3)
    qy = jnp.pad(pos_q[:, 1], (0, qpad), constant_values=2e3)
    qz = jnp.pad(pos_q[:, 2], (0, qpad), constant_values=2e3)
    iq = jnp.pad(idx, (0, qpad))

    nbr_flat = _ballq(px, py, pz, qx, qy, qz, iq)
    xg, rx, ry, rz = _gather(x, nbr_flat, px, py, pz, qx, qy, qz)
    out = _conv(xg, rx, ry, rz, W1, b1, W2, b2)[:_M]
    return (out, pos_q, batch[idx])


# speedup vs baseline: 16.6522x; 1.1573x over previous
import functools

import jax
import jax.numpy as jnp
from jax import lax
from jax.experimental import pallas as pl
from jax.experimental.pallas import tpu as pltpu
from jax.experimental.pallas import tpu_sc as plsc

_N = 10000
_M = 2500
_R2 = 0.2 * 0.2
_K = 64
_NPAD = 10240
_ROWS = 80
_MPAD = 2560
_NW = 32
_RW = _MPAD // _NW
_CAP = 768
_BIGBITS = 0x7F700000
_R2BITS = __import__("struct").unpack("<i", __import__("struct").pack("<f", _R2))[0]


def _sc_mesh():
    return plsc.VectorSubcoreMesh(
        core_axis_name="c", subcore_axis_name="s", num_cores=2, num_subcores=16)


def _fps_body(px_ref, py_ref, pz_ref, pxs_ref, pys_ref, pzs_ref, mind0_ref,
              out_ref, mind_s):
    mind_s[...] = mind0_ref[...]
    flat = (lax.broadcasted_iota(jnp.int32, (_ROWS, 128), 0) * 128
            + lax.broadcasted_iota(jnp.int32, (_ROWS, 128), 1))
    out_ref[0] = 0

    def body(i, _):
        px = px_ref[...]
        py = py_ref[...]
        pz = pz_ref[...]
        mind = mind_s[...]
        m = jnp.max(mind)
        eq = mind == m
        nxt = jnp.min(jnp.where(eq, flat, jnp.int32(2**30)))
        qx = pxs_ref[nxt]
        qy = pys_ref[nxt]
        qz = pzs_ref[nxt]
        dx = px - qx
        dy = py - qy
        dz = pz - qz
        d = dx * dx + (dy * dy + dz * dz)
        mind_s[...] = jnp.minimum(mind, d)
        out_ref[i] = nxt
        return 0

    lax.fori_loop(1, _M, body, 0)


def _fps(pos):
    pad = _NPAD - _N
    pxf = jnp.pad(pos[:, 0], (0, pad))
    pyf = jnp.pad(pos[:, 1], (0, pad))
    pzf = jnp.pad(pos[:, 2], (0, pad))
    px = pxf.reshape(_ROWS, 128)
    py = pyf.reshape(_ROWS, 128)
    pz = pzf.reshape(_ROWS, 128)
    d0 = jnp.sum((pos - pos[0]) ** 2, axis=1)
    mind0 = jnp.pad(d0, (0, pad), constant_values=-1.0).reshape(_ROWS, 128)
    return pl.pallas_call(
        _fps_body,
        out_shape=jax.ShapeDtypeStruct((_M,), jnp.int32),
        in_specs=[pl.BlockSpec(memory_space=pltpu.VMEM)] * 3
        + [pl.BlockSpec(memory_space=pltpu.SMEM)] * 3
        + [pl.BlockSpec(memory_space=pltpu.VMEM)],
        out_specs=pl.BlockSpec(memory_space=pltpu.SMEM),
        scratch_shapes=[pltpu.VMEM((_ROWS, 128), jnp.float32)],
    )(px, py, pz, pxf, pyf, pzf, mind0)


def _ballq_body(px_hbm, py_hbm, pz_hbm, qx_hbm, qy_hbm, qz_hbm, iq_hbm,
                nbr_hbm, px_v, py_v, pz_v, qx_v, qy_v, qz_v, iq_v,
                cand_v, bits_v, eq_v, sel_v, sem):
    wid = lax.axis_index("s") * 2 + lax.axis_index("c")
    rbase = wid * _RW
    pltpu.sync_copy(px_hbm.at[pl.ds(0, _NPAD)], px_v)
    pltpu.sync_copy(py_hbm.at[pl.ds(0, _NPAD)], py_v)
    pltpu.sync_copy(pz_hbm.at[pl.ds(0, _NPAD)], pz_v)
    pltpu.sync_copy(qx_hbm.at[pl.ds(rbase, _RW)], qx_v)
    pltpu.sync_copy(qy_hbm.at[pl.ds(rbase, _RW)], qy_v)
    pltpu.sync_copy(qz_hbm.at[pl.ds(rbase, _RW)], qz_v)
    pltpu.sync_copy(iq_hbm.at[pl.ds(rbase, _RW)], iq_v)

    iota = lax.iota(jnp.int32, 16)
    lane_base = iota * _CAP

    def group(g, _):
        qxg = qx_v[pl.ds(g * 16, 16)]
        qyg = qy_v[pl.ds(g * 16, 16)]
        qzg = qz_v[pl.ds(g * 16, 16)]

        def scan(c, off):
            lx = px_v[pl.ds(c * 16, 16)]
            ly = py_v[pl.ds(c * 16, 16)]
            lz = pz_v[pl.ds(c * 16, 16)]
            for l in range(16):
                sp = jnp.full((16,), l, jnp.int32)
                dx = jnp.take(lx, sp) - qxg
                dy = jnp.take(ly, sp) - qyg
                dz = jnp.take(lz, sp) - qzg
                d = dx * dx + (dy * dy + dz * dz)
                mm = d < _R2
                tgt = lane_base + jnp.minimum(off, _CAP - 1)
                plsc.store_scatter(cand_v, [tgt], c * 16 + l + jnp.zeros((16,), jnp.int32), mask=mm)
                off = off + mm.astype(jnp.int32)
            return off

        off = lax.fori_loop(0, _NPAD // 16, scan, jnp.zeros((16,), jnp.int32))

        for l in range(16):
            rloc = g * 16 + l
            cnt = jnp.minimum(off[l], _CAP - 1)
            k_take = jnp.minimum(cnt, _K)
            nc = (cnt + 15) // 16
            qxb = jnp.take(qxg, jnp.full((16,), l, jnp.int32))
            qyb = jnp.take(qyg, jnp.full((16,), l, jnp.int32))
            qzb = jnp.take(qzg, jnp.full((16,), l, jnp.int32))

            def rebuild(ch, _):
                ci = cand_v[pl.ds(l * _CAP + ch * 16, 16)]
                ci = jnp.clip(ci, 0, _NPAD - 1)
                gx = plsc.load_gather(px_v, [ci]) - qxb
                gy = plsc.load_gather(py_v, [ci]) - qyb
                gz = plsc.load_gather(pz_v, [ci]) - qzb
                d = gx * gx + (gy * gy + gz * gz)
                db = plsc.bitcast(d, jnp.int32)
                lane = ch * 16 + iota
                db = jnp.where(lane < cnt, db, _BIGBITS)
                bits_v[pl.ds(ch * 16, 16)] = db
                return 0

            lax.fori_loop(0, nc, rebuild, 0)

            def count_le(t):
                def cbody(ch, acc):
                    db = bits_v[pl.ds(ch * 16, 16)]
                    return acc + plsc.all_reduce_population_count(db <= t)

                acc = lax.fori_loop(0, nc, cbody, jnp.zeros((16,), jnp.int32))
                return acc[0]

            def bis(_, lohi):
                lo, hi = lohi
                mid = lo + (hi - lo) // 2
                c = count_le(mid)
                return jnp.where(c >= k_take, lo, mid), jnp.where(c >= k_take, mid, hi)

            lo, hi = lax.fori_loop(0, 31, bis, (jnp.int32(-1), jnp.int32(_R2BITS)))
            t_star = hi
            c_lt = count_le(t_star - 1)
            need_eq = k_take - c_lt

            self_sp = plsc.load_gather(iq_v, [jnp.full((16,), rloc, jnp.int32)])
            for ch4 in range(4):
                sel_v[pl.ds(rloc * _K + ch4 * 16, 16)] = self_sp

            def emit(ch, offs):
                o_lt, o_eq = offs
                db = bits_v[pl.ds(ch * 16, 16)]
                ci = cand_v[pl.ds(l * _CAP + ch * 16, 16)]
                m_lt = db < t_star
                m_eq = db == t_star
                plsc.store_compressed(sel_v.at[pl.ds(rloc * _K + o_lt, 16)], ci, mask=m_lt)
                plsc.store_compressed(eq_v.at[pl.ds(o_eq, 16)], ci, mask=m_eq)
                return (o_lt + plsc.all_reduce_population_count(m_lt)[0],
                        o_eq + plsc.all_reduce_population_count(m_eq)[0])

            lax.fori_loop(0, nc, emit, (jnp.int32(0), jnp.int32(0)))

            for ch4 in range(4):
                mpref = (ch4 * 16 + iota) < need_eq
                ev = eq_v[pl.ds(ch4 * 16, 16)]
                o3 = rloc * _K + jnp.minimum(c_lt + ch4 * 16, _K)
                plsc.store_compressed(sel_v.at[pl.ds(o3, 16)], ev, mask=mpref)
        return 0

    lax.fori_loop(0, _RW // 16, group, 0)
    pltpu.sync_copy(sel_v.at[pl.ds(0, _RW * _K)], nbr_hbm.at[pl.ds(rbase * _K, _RW * _K)])


def _ballq(px, py, pz, qx, qy, qz, iq):
    f = functools.partial(
        pl.kernel,
        out_type=jax.ShapeDtypeStruct((_MPAD * _K,), jnp.int32),
        mesh=_sc_mesh(),
        scratch_types=[
            pltpu.VMEM((_NPAD,), jnp.float32),
            pltpu.VMEM((_NPAD,), jnp.float32),
            pltpu.VMEM((_NPAD,), jnp.float32),
            pltpu.VMEM((_RW,), jnp.float32),
            pltpu.VMEM((_RW,), jnp.float32),
            pltpu.VMEM((_RW,), jnp.float32),
            pltpu.VMEM((_RW,), jnp.int32),
            pltpu.VMEM((16 * _CAP,), jnp.int32),
            pltpu.VMEM((_CAP + 16,), jnp.int32),
            pltpu.VMEM((_CAP + 16,), jnp.int32),
            pltpu.VMEM((_RW * _K + 16,), jnp.int32),
            pltpu.SemaphoreType.DMA,
        ],
        compiler_params=pltpu.CompilerParams(needs_layout_passes=False),
    )(_ballq_body)
    return f(px, py, pz, qx, qy, qz, iq)


_GC = 512


def _gather_body(x_hbm, nbr_hbm, px_hbm, py_hbm, pz_hbm, qx_hbm, qy_hbm, qz_hbm,
                 xg_hbm, rx_hbm, ry_hbm, rz_hbm,
                 px_v, py_v, pz_v, qx_v, qy_v, qz_v, idx_v, rows_v,
                 rx_v, ry_v, rz_v, sem):
    wid = lax.axis_index("s") * 2 + lax.axis_index("c")
    rbase = wid * _RW
    fbase = rbase * _K
    pltpu.sync_copy(px_hbm.at[pl.ds(0, _NPAD)], px_v)
    pltpu.sync_copy(py_hbm.at[pl.ds(0, _NPAD)], py_v)
    pltpu.sync_copy(pz_hbm.at[pl.ds(0, _NPAD)], pz_v)
    pltpu.sync_copy(qx_hbm.at[pl.ds(rbase, _RW)], qx_v)
    pltpu.sync_copy(qy_hbm.at[pl.ds(rbase, _RW)], qy_v)
    pltpu.sync_copy(qz_hbm.at[pl.ds(rbase, _RW)], qz_v)

    def chunk(ch, _):
        pltpu.sync_copy(nbr_hbm.at[pl.ds(fbase + ch * _GC, _GC)], idx_v)
        cp = pltpu.async_copy(x_hbm.at[idx_v], rows_v, sem)

        def sub(sc, _):
            ci = idx_v[pl.ds(sc * 16, 16)]
            rloc = ch * (_GC // _K) + sc // 4
            sp = jnp.full((16,), rloc, jnp.int32)
            gx = plsc.load_gather(px_v, [ci]) - plsc.load_gather(qx_v, [sp])
            gy = plsc.load_gather(py_v, [ci]) - plsc.load_gather(qy_v, [sp])
            gz = plsc.load_gather(pz_v, [ci]) - plsc.load_gather(qz_v, [sp])
            rx_v[pl.ds(sc * 16, 16)] = gx
            ry_v[pl.ds(sc * 16, 16)] = gy
            rz_v[pl.ds(sc * 16, 16)] = gz
            return 0

        lax.fori_loop(0, _GC // 16, sub, 0)
        pltpu.sync_copy(rx_v, rx_hbm.at[pl.ds(fbase + ch * _GC, _GC)])
        pltpu.sync_copy(ry_v, ry_hbm.at[pl.ds(fbase + ch * _GC, _GC)])
        pltpu.sync_copy(rz_v, rz_hbm.at[pl.ds(fbase + ch * _GC, _GC)])
        cp.wait()
        pltpu.sync_copy(rows_v, xg_hbm.at[pl.ds(fbase + ch * _GC, _GC)])
        return 0

    lax.fori_loop(0, _RW * _K // _GC, chunk, 0)


def _gather(x, nbr_flat, px, py, pz, qx, qy, qz):
    f = functools.partial(
        pl.kernel,
        out_type=(
            jax.ShapeDtypeStruct((_MPAD * _K, 128), jnp.float32),
            jax.ShapeDtypeStruct((_MPAD * _K,), jnp.float32),
            jax.ShapeDtypeStruct((_MPAD * _K,), jnp.float32),
            jax.ShapeDtypeStruct((_MPAD * _K,), jnp.float32),
        ),
        mesh=_sc_mesh(),
        scratch_types=[
            pltpu.VMEM((_NPAD,), jnp.float32),
            pltpu.VMEM((_NPAD,), jnp.float32),
            pltpu.VMEM((_NPAD,), jnp.float32),
            pltpu.VMEM((_RW,), jnp.float32),
            pltpu.VMEM((_RW,), jnp.float32),
            pltpu.VMEM((_RW,), jnp.float32),
            pltpu.VMEM((_GC,), jnp.int32),
            pltpu.VMEM((_GC, 128), jnp.float32),
            pltpu.VMEM((_GC,), jnp.float32),
            pltpu.VMEM((_GC,), jnp.float32),
            pltpu.VMEM((_GC,), jnp.float32),
            pltpu.SemaphoreType.DMA,
        ],
        compiler_params=pltpu.CompilerParams(needs_layout_passes=False),
    )(_gather_body)
    return f(x, nbr_flat, px, py, pz, qx, qy, qz)


_CB = 64


def _conv_body(xj_ref, rx_ref, ry_ref, rz_ref, w1x_ref, p0_ref, p1_ref, p2_ref,
               b1_ref, w2_ref, b2_ref, out_ref):
    h = jnp.dot(xj_ref[...], w1x_ref[...], preferred_element_type=jnp.float32)
    h = h + rx_ref[...] * p0_ref[...]
    h = h + ry_ref[...] * p1_ref[...]
    h = h + rz_ref[...] * p2_ref[...]
    h = jnp.maximum(h + b1_ref[...], 0.0)
    h = jnp.dot(h, w2_ref[...], preferred_element_type=jnp.float32) + b2_ref[...]
    h = jnp.maximum(h, 0.0)
    out_ref[...] = jnp.max(h.reshape(_CB, _K, 128), axis=1)


def _conv(xj, rx, ry, rz, W1, b1, W2, b2):
    w1x = W1[:128]
    grid = _MPAD // _CB
    full = lambda i: (0, 0)
    return pl.pallas_call(
        _conv_body,
        grid=(grid,),
        in_specs=[
            pl.BlockSpec((_CB * _K, 128), lambda i: (i, 0)),
            pl.BlockSpec((_CB * _K, 1), lambda i: (i, 0)),
            pl.BlockSpec((_CB * _K, 1), lambda i: (i, 0)),
            pl.BlockSpec((_CB * _K, 1), lambda i: (i, 0)),
            pl.BlockSpec((128, 128), full),
            pl.BlockSpec((1, 128), full),
            pl.BlockSpec((1, 128), full),
            pl.BlockSpec((1, 128), full),
            pl.BlockSpec((1, 128), full),
            pl.BlockSpec((128, 128), full),
            pl.BlockSpec((1, 128), full),
        ],
        out_specs=pl.BlockSpec((_CB, 128), lambda i: (i, 0)),
        out_shape=jax.ShapeDtypeStruct((_MPAD, 128), jnp.float32),
    )(xj, rx.reshape(-1, 1), ry.reshape(-1, 1), rz.reshape(-1, 1),
      w1x, W1[128].reshape(1, 128), W1[129].reshape(1, 128), W1[130].reshape(1, 128),
      b1.reshape(1, 128), W2, b2.reshape(1, 128))


def kernel(x, pos, batch, W1, b1, W2, b2):
    idx = _fps(pos)
    pos_q = pos[idx]

    pad = _NPAD - _N
    px = jnp.pad(pos[:, 0], (0, pad), constant_values=1e3)
    py = jnp.pad(pos[:, 1], (0, pad), constant_values=1e3)
    pz = jnp.pad(pos[:, 2], (0, pad), constant_values=1e3)
    qpad = _MPAD - _M
    qx = jnp.pad(pos_q[:, 0], (0, qpad), constant_values=2e3)
    qy = jnp.pad(pos_q[:, 1], (0, qpad), constant_values=2e3)
    qz = jnp.pad(pos_q[:, 2], (0, qpad), constant_values=2e3)
    iq = jnp.pad(idx, (0, qpad))

    nbr_flat = _ballq(px, py, pz, qx, qy, qz, iq)
    xg, rx, ry, rz = _gather(x, nbr_flat, px, py, pz, qx, qy, qz)
    out = _conv(xg, rx, ry, rz, W1, b1, W2, b2)[:_M]
    return (out, pos_q, batch[idx])


# speedup vs baseline: 16.8594x; 1.0124x over previous
import functools

import jax
import jax.numpy as jnp
from jax import lax
from jax.experimental import pallas as pl
from jax.experimental.pallas import tpu as pltpu
from jax.experimental.pallas import tpu_sc as plsc

_N = 10000
_M = 2500
_R2 = 0.2 * 0.2
_K = 64
_NPAD = 10240
_ROWS = 80
_MPAD = 2560
_NW = 32
_RW = _MPAD // _NW
_CAP = 768
_BIGBITS = 0x7F700000
_R2BITS = __import__("struct").unpack("<i", __import__("struct").pack("<f", _R2))[0]


def _sc_mesh():
    return plsc.VectorSubcoreMesh(
        core_axis_name="c", subcore_axis_name="s", num_cores=2, num_subcores=16)


def _fps_body(px_ref, py_ref, pz_ref, pxs_ref, pys_ref, pzs_ref, mind0_ref,
              out_ref, mind_s):
    mind_s[...] = mind0_ref[...]
    flat = (lax.broadcasted_iota(jnp.int32, (_ROWS, 128), 0) * 128
            + lax.broadcasted_iota(jnp.int32, (_ROWS, 128), 1))
    out_ref[0] = 0

    def body(i, _):
        px = px_ref[...]
        py = py_ref[...]
        pz = pz_ref[...]
        mind = mind_s[...]
        m = jnp.max(mind)
        eq = mind == m
        nxt = jnp.min(jnp.where(eq, flat, jnp.int32(2**30)))
        qx = pxs_ref[nxt]
        qy = pys_ref[nxt]
        qz = pzs_ref[nxt]
        dx = px - qx
        dy = py - qy
        dz = pz - qz
        d = dx * dx + (dy * dy + dz * dz)
        mind_s[...] = jnp.minimum(mind, d)
        out_ref[i] = nxt
        return 0

    lax.fori_loop(1, _M, body, 0)


def _fps(pos):
    pad = _NPAD - _N
    pxf = jnp.pad(pos[:, 0], (0, pad))
    pyf = jnp.pad(pos[:, 1], (0, pad))
    pzf = jnp.pad(pos[:, 2], (0, pad))
    px = pxf.reshape(_ROWS, 128)
    py = pyf.reshape(_ROWS, 128)
    pz = pzf.reshape(_ROWS, 128)
    d0 = jnp.sum((pos - pos[0]) ** 2, axis=1)
    mind0 = jnp.pad(d0, (0, pad), constant_values=-1.0).reshape(_ROWS, 128)
    return pl.pallas_call(
        _fps_body,
        out_shape=jax.ShapeDtypeStruct((_M,), jnp.int32),
        in_specs=[pl.BlockSpec(memory_space=pltpu.VMEM)] * 3
        + [pl.BlockSpec(memory_space=pltpu.SMEM)] * 3
        + [pl.BlockSpec(memory_space=pltpu.VMEM)],
        out_specs=pl.BlockSpec(memory_space=pltpu.SMEM),
        scratch_shapes=[pltpu.VMEM((_ROWS, 128), jnp.float32)],
    )(px, py, pz, pxf, pyf, pzf, mind0)


def _ballq_body(px_hbm, py_hbm, pz_hbm, qx_hbm, qy_hbm, qz_hbm, iq_hbm,
                nbr_hbm, px_v, py_v, pz_v, qx_v, qy_v, qz_v, iq_v,
                cand_v, bits_v, eq_v, sel_v, sem):
    wid = lax.axis_index("s") * 2 + lax.axis_index("c")
    rbase = wid * _RW
    pltpu.sync_copy(px_hbm.at[pl.ds(0, _NPAD)], px_v)
    pltpu.sync_copy(py_hbm.at[pl.ds(0, _NPAD)], py_v)
    pltpu.sync_copy(pz_hbm.at[pl.ds(0, _NPAD)], pz_v)
    pltpu.sync_copy(qx_hbm.at[pl.ds(rbase, _RW)], qx_v)
    pltpu.sync_copy(qy_hbm.at[pl.ds(rbase, _RW)], qy_v)
    pltpu.sync_copy(qz_hbm.at[pl.ds(rbase, _RW)], qz_v)
    pltpu.sync_copy(iq_hbm.at[pl.ds(rbase, _RW)], iq_v)

    iota = lax.iota(jnp.int32, 16)
    lane_base = iota * _CAP

    def group(g, _):
        qxg = qx_v[pl.ds(g * 16, 16)]
        qyg = qy_v[pl.ds(g * 16, 16)]
        qzg = qz_v[pl.ds(g * 16, 16)]

        def scan(c, off):
            lx = px_v[pl.ds(c * 16, 16)]
            ly = py_v[pl.ds(c * 16, 16)]
            lz = pz_v[pl.ds(c * 16, 16)]
            for l in range(16):
                sp = jnp.full((16,), l, jnp.int32)
                dx = jnp.take(lx, sp) - qxg
                dy = jnp.take(ly, sp) - qyg
                dz = jnp.take(lz, sp) - qzg
                d = dx * dx + (dy * dy + dz * dz)
                mm = d < _R2
                tgt = lane_base + jnp.minimum(off, _CAP - 1)
                plsc.store_scatter(cand_v, [tgt], c * 16 + l + jnp.zeros((16,), jnp.int32), mask=mm)
                off = off + mm.astype(jnp.int32)
            return off

        off = lax.fori_loop(0, _NPAD // 16, scan, jnp.zeros((16,), jnp.int32))

        for l in range(16):
            rloc = g * 16 + l
            cnt = jnp.minimum(off[l], _CAP - 1)
            k_take = jnp.minimum(cnt, _K)
            nc = (cnt + 15) // 16
            qxb = jnp.take(qxg, jnp.full((16,), l, jnp.int32))
            qyb = jnp.take(qyg, jnp.full((16,), l, jnp.int32))
            qzb = jnp.take(qzg, jnp.full((16,), l, jnp.int32))

            def rebuild(ch, _):
                ci = cand_v[pl.ds(l * _CAP + ch * 16, 16)]
                ci = jnp.clip(ci, 0, _NPAD - 1)
                gx = plsc.load_gather(px_v, [ci]) - qxb
                gy = plsc.load_gather(py_v, [ci]) - qyb
                gz = plsc.load_gather(pz_v, [ci]) - qzb
                d = gx * gx + (gy * gy + gz * gz)
                db = plsc.bitcast(d, jnp.int32)
                lane = ch * 16 + iota
                db = jnp.where(lane < cnt, db, _BIGBITS)
                bits_v[pl.ds(ch * 16, 16)] = db
                return 0

            lax.fori_loop(0, nc, rebuild, 0)

            def count_le(t):
                def cbody(ch, acc):
                    db = bits_v[pl.ds(ch * 16, 16)]
                    return acc + plsc.all_reduce_population_count(db <= t)

                acc = lax.fori_loop(0, nc, cbody, jnp.zeros((16,), jnp.int32))
                return acc[0]

            def bis(_, lohi):
                lo, hi = lohi
                mid = lo + (hi - lo) // 2
                c = count_le(mid)
                return jnp.where(c >= k_take, lo, mid), jnp.where(c >= k_take, mid, hi)

            lo, hi = lax.fori_loop(0, 31, bis, (jnp.int32(-1), jnp.int32(_R2BITS)))
            t_star = hi
            c_lt = count_le(t_star - 1)
            need_eq = k_take - c_lt

            self_sp = plsc.load_gather(iq_v, [jnp.full((16,), rloc, jnp.int32)])
            for ch4 in range(4):
                sel_v[pl.ds(rloc * _K + ch4 * 16, 16)] = self_sp

            def emit(ch, offs):
                o_lt, o_eq = offs
                db = bits_v[pl.ds(ch * 16, 16)]
                ci = cand_v[pl.ds(l * _CAP + ch * 16, 16)]
                m_lt = db < t_star
                m_eq = db == t_star
                plsc.store_compressed(sel_v.at[pl.ds(rloc * _K + o_lt, 16)], ci, mask=m_lt)
                plsc.store_compressed(eq_v.at[pl.ds(o_eq, 16)], ci, mask=m_eq)
                return (o_lt + plsc.all_reduce_population_count(m_lt)[0],
                        o_eq + plsc.all_reduce_population_count(m_eq)[0])

            lax.fori_loop(0, nc, emit, (jnp.int32(0), jnp.int32(0)))

            for ch4 in range(4):
                mpref = (ch4 * 16 + iota) < need_eq
                ev = eq_v[pl.ds(ch4 * 16, 16)]
                o3 = rloc * _K + jnp.minimum(c_lt + ch4 * 16, _K)
                plsc.store_compressed(sel_v.at[pl.ds(o3, 16)], ev, mask=mpref)
        return 0

    lax.fori_loop(0, _RW // 16, group, 0)
    pltpu.sync_copy(sel_v.at[pl.ds(0, _RW * _K)], nbr_hbm.at[pl.ds(rbase * _K, _RW * _K)])


def _ballq(px, py, pz, qx, qy, qz, iq):
    f = functools.partial(
        pl.kernel,
        out_type=jax.ShapeDtypeStruct((_MPAD * _K,), jnp.int32),
        mesh=_sc_mesh(),
        scratch_types=[
            pltpu.VMEM((_NPAD,), jnp.float32),
            pltpu.VMEM((_NPAD,), jnp.float32),
            pltpu.VMEM((_NPAD,), jnp.float32),
            pltpu.VMEM((_RW,), jnp.float32),
            pltpu.VMEM((_RW,), jnp.float32),
            pltpu.VMEM((_RW,), jnp.float32),
            pltpu.VMEM((_RW,), jnp.int32),
            pltpu.VMEM((16 * _CAP,), jnp.int32),
            pltpu.VMEM((_CAP + 16,), jnp.int32),
            pltpu.VMEM((_CAP + 16,), jnp.int32),
            pltpu.VMEM((_RW * _K + 16,), jnp.int32),
            pltpu.SemaphoreType.DMA,
        ],
        compiler_params=pltpu.CompilerParams(needs_layout_passes=False),
    )(_ballq_body)
    return f(px, py, pz, qx, qy, qz, iq)


_GC = 256
_NCH = _RW * _K // _GC


def _gather_body(x_hbm, nbr_hbm, px_hbm, py_hbm, pz_hbm, qx_hbm, qy_hbm, qz_hbm,
                 xg_hbm, rx_hbm, ry_hbm, rz_hbm,
                 px_v, py_v, pz_v, qx_v, qy_v, qz_v,
                 idx_a, idx_b, rows_a, rows_b,
                 rx_v, ry_v, rz_v, sem_a, sem_b):
    wid = lax.axis_index("s") * 2 + lax.axis_index("c")
    rbase = wid * _RW
    fbase = rbase * _K
    pltpu.sync_copy(px_hbm.at[pl.ds(0, _NPAD)], px_v)
    pltpu.sync_copy(py_hbm.at[pl.ds(0, _NPAD)], py_v)
    pltpu.sync_copy(pz_hbm.at[pl.ds(0, _NPAD)], pz_v)
    pltpu.sync_copy(qx_hbm.at[pl.ds(rbase, _RW)], qx_v)
    pltpu.sync_copy(qy_hbm.at[pl.ds(rbase, _RW)], qy_v)
    pltpu.sync_copy(qz_hbm.at[pl.ds(rbase, _RW)], qz_v)

    def rel_and_out(ch, idx_v, rows_v, sem):
        def sub(sc, _):
            ci = idx_v[pl.ds(sc * 16, 16)]
            rloc = ch * (_GC // _K) + sc // 4
            sp = jnp.full((16,), rloc, jnp.int32)
            gx = plsc.load_gather(px_v, [ci]) - plsc.load_gather(qx_v, [sp])
            gy = plsc.load_gather(py_v, [ci]) - plsc.load_gather(qy_v, [sp])
            gz = plsc.load_gather(pz_v, [ci]) - plsc.load_gather(qz_v, [sp])
            rx_v[pl.ds(sc * 16, 16)] = gx
            ry_v[pl.ds(sc * 16, 16)] = gy
            rz_v[pl.ds(sc * 16, 16)] = gz
            return 0

        lax.fori_loop(0, _GC // 16, sub, 0)
        pltpu.sync_copy(rx_v, rx_hbm.at[pl.ds(fbase + ch * _GC, _GC)])
        pltpu.sync_copy(ry_v, ry_hbm.at[pl.ds(fbase + ch * _GC, _GC)])
        pltpu.sync_copy(rz_v, rz_hbm.at[pl.ds(fbase + ch * _GC, _GC)])
        pltpu.make_async_copy(x_hbm.at[idx_v], rows_v, sem).wait()
        pltpu.sync_copy(rows_v, xg_hbm.at[pl.ds(fbase + ch * _GC, _GC)])

    pltpu.sync_copy(nbr_hbm.at[pl.ds(fbase, _GC)], idx_a)
    pltpu.async_copy(x_hbm.at[idx_a], rows_a, sem_a)

    def pair(i, _):
        chb = 2 * i + 1
        pltpu.sync_copy(nbr_hbm.at[pl.ds(fbase + chb * _GC, _GC)], idx_b)
        pltpu.async_copy(x_hbm.at[idx_b], rows_b, sem_b)
        rel_and_out(2 * i, idx_a, rows_a, sem_a)

        @pl.when(i < _NCH // 2 - 1)
        def _():
            cha = 2 * i + 2
            pltpu.sync_copy(nbr_hbm.at[pl.ds(fbase + cha * _GC, _GC)], idx_a)
            pltpu.async_copy(x_hbm.at[idx_a], rows_a, sem_a)

        rel_and_out(chb, idx_b, rows_b, sem_b)
        return 0

    lax.fori_loop(0, _NCH // 2, pair, 0)


def _gather(x, nbr_flat, px, py, pz, qx, qy, qz):
    f = functools.partial(
        pl.kernel,
        out_type=(
            jax.ShapeDtypeStruct((_MPAD * _K, 128), jnp.float32),
            jax.ShapeDtypeStruct((_MPAD * _K,), jnp.float32),
            jax.ShapeDtypeStruct((_MPAD * _K,), jnp.float32),
            jax.ShapeDtypeStruct((_MPAD * _K,), jnp.float32),
        ),
        mesh=_sc_mesh(),
        scratch_types=[
            pltpu.VMEM((_NPAD,), jnp.float32),
            pltpu.VMEM((_NPAD,), jnp.float32),
            pltpu.VMEM((_NPAD,), jnp.float32),
            pltpu.VMEM((_RW,), jnp.float32),
            pltpu.VMEM((_RW,), jnp.float32),
            pltpu.VMEM((_RW,), jnp.float32),
            pltpu.VMEM((_GC,), jnp.int32),
            pltpu.VMEM((_GC,), jnp.int32),
            pltpu.VMEM((_GC, 128), jnp.float32),
            pltpu.VMEM((_GC, 128), jnp.float32),
            pltpu.VMEM((_GC,), jnp.float32),
            pltpu.VMEM((_GC,), jnp.float32),
            pltpu.VMEM((_GC,), jnp.float32),
            pltpu.SemaphoreType.DMA,
            pltpu.SemaphoreType.DMA,
        ],
        compiler_params=pltpu.CompilerParams(needs_layout_passes=False),
    )(_gather_body)
    return f(x, nbr_flat, px, py, pz, qx, qy, qz)


_CB = 64


def _conv_body(xj_ref, rx_ref, ry_ref, rz_ref, w1x_ref, p0_ref, p1_ref, p2_ref,
               b1_ref, w2_ref, b2_ref, out_ref):
    h = jnp.dot(xj_ref[...], w1x_ref[...], preferred_element_type=jnp.float32)
    h = h + rx_ref[...] * p0_ref[...]
    h = h + ry_ref[...] * p1_ref[...]
    h = h + rz_ref[...] * p2_ref[...]
    h = jnp.maximum(h + b1_ref[...], 0.0)
    h = jnp.dot(h, w2_ref[...], preferred_element_type=jnp.float32) + b2_ref[...]
    h = jnp.maximum(h, 0.0)
    out_ref[...] = jnp.max(h.reshape(_CB, _K, 128), axis=1)


def _conv(xj, rx, ry, rz, W1, b1, W2, b2):
    w1x = W1[:128]
    grid = _MPAD // _CB
    full = lambda i: (0, 0)
    return pl.pallas_call(
        _conv_body,
        grid=(grid,),
        in_specs=[
            pl.BlockSpec((_CB * _K, 128), lambda i: (i, 0)),
            pl.BlockSpec((_CB * _K, 1), lambda i: (i, 0)),
            pl.BlockSpec((_CB * _K, 1), lambda i: (i, 0)),
            pl.BlockSpec((_CB * _K, 1), lambda i: (i, 0)),
            pl.BlockSpec((128, 128), full),
            pl.BlockSpec((1, 128), full),
            pl.BlockSpec((1, 128), full),
            pl.BlockSpec((1, 128), full),
            pl.BlockSpec((1, 128), full),
            pl.BlockSpec((128, 128), full),
            pl.BlockSpec((1, 128), full),
        ],
        out_specs=pl.BlockSpec((_CB, 128), lambda i: (i, 0)),
        out_shape=jax.ShapeDtypeStruct((_MPAD, 128), jnp.float32),
    )(xj, rx.reshape(-1, 1), ry.reshape(-1, 1), rz.reshape(-1, 1),
      w1x, W1[128].reshape(1, 128), W1[129].reshape(1, 128), W1[130].reshape(1, 128),
      b1.reshape(1, 128), W2, b2.reshape(1, 128))


def kernel(x, pos, batch, W1, b1, W2, b2):
    idx = _fps(pos)
    pos_q = pos[idx]

    pad = _NPAD - _N
    px = jnp.pad(pos[:, 0], (0, pad), constant_values=1e3)
    py = jnp.pad(pos[:, 1], (0, pad), constant_values=1e3)
    pz = jnp.pad(pos[:, 2], (0, pad), constant_values=1e3)
    qpad = _MPAD - _M
    qx = jnp.pad(pos_q[:, 0], (0, qpad), constant_values=2e3)
    qy = jnp.pad(pos_q[:, 1], (0, qpad), constant_values=2e3)
    qz = jnp.pad(pos_q[:, 2], (0, qpad), constant_values=2e3)
    iq = jnp.pad(idx, (0, qpad))

    nbr_flat = _ballq(px, py, pz, qx, qy, qz, iq)
    xg, rx, ry, rz = _gather(x, nbr_flat, px, py, pz, qx, qy, qz)
    out = _conv(xg, rx, ry, rz, W1, b1, W2, b2)[:_M]
    return (out, pos_q, batch[idx])


# speedup vs baseline: 17.0632x; 1.0121x over previous
import functools

import jax
import jax.numpy as jnp
from jax import lax
from jax.experimental import pallas as pl
from jax.experimental.pallas import tpu as pltpu
from jax.experimental.pallas import tpu_sc as plsc

_N = 10000
_M = 2500
_R2 = 0.2 * 0.2
_K = 64
_NPAD = 10240
_ROWS = 80
_MPAD = 2560
_NW = 32
_RW = _MPAD // _NW
_CAP = 768
_BIGBITS = 0x7F700000
_R2BITS = __import__("struct").unpack("<i", __import__("struct").pack("<f", _R2))[0]


def _sc_mesh():
    return plsc.VectorSubcoreMesh(
        core_axis_name="c", subcore_axis_name="s", num_cores=2, num_subcores=16)


def _fps_body(px_ref, py_ref, pz_ref, pxs_ref, pys_ref, pzs_ref, mind0_ref,
              out_ref, mind_s):
    del mind_s
    flat = (lax.broadcasted_iota(jnp.int32, (_ROWS, 128), 0) * 128
            + lax.broadcasted_iota(jnp.int32, (_ROWS, 128), 1))
    out_ref[0] = 0
    px = px_ref[...]
    py = py_ref[...]
    pz = pz_ref[...]

    def body(i, mind):
        m = jnp.max(mind)
        eq = mind == m
        nxt = jnp.min(jnp.where(eq, flat, jnp.int32(2**30)))
        qx = pxs_ref[nxt]
        qy = pys_ref[nxt]
        qz = pzs_ref[nxt]
        dx = px - qx
        dy = py - qy
        dz = pz - qz
        d = dx * dx + (dy * dy + dz * dz)
        out_ref[i] = nxt
        return jnp.minimum(mind, d)

    lax.fori_loop(1, _M, body, mind0_ref[...])


def _fps(pos):
    pad = _NPAD - _N
    pxf = jnp.pad(pos[:, 0], (0, pad))
    pyf = jnp.pad(pos[:, 1], (0, pad))
    pzf = jnp.pad(pos[:, 2], (0, pad))
    px = pxf.reshape(_ROWS, 128)
    py = pyf.reshape(_ROWS, 128)
    pz = pzf.reshape(_ROWS, 128)
    d0 = jnp.sum((pos - pos[0]) ** 2, axis=1)
    mind0 = jnp.pad(d0, (0, pad), constant_values=-1.0).reshape(_ROWS, 128)
    return pl.pallas_call(
        _fps_body,
        out_shape=jax.ShapeDtypeStruct((_M,), jnp.int32),
        in_specs=[pl.BlockSpec(memory_space=pltpu.VMEM)] * 3
        + [pl.BlockSpec(memory_space=pltpu.SMEM)] * 3
        + [pl.BlockSpec(memory_space=pltpu.VMEM)],
        out_specs=pl.BlockSpec(memory_space=pltpu.SMEM),
        scratch_shapes=[pltpu.VMEM((_ROWS, 128), jnp.float32)],
    )(px, py, pz, pxf, pyf, pzf, mind0)


def _ballq_body(px_hbm, py_hbm, pz_hbm, qx_hbm, qy_hbm, qz_hbm, iq_hbm,
                nbr_hbm, px_v, py_v, pz_v, qx_v, qy_v, qz_v, iq_v,
                cand_v, bits_v, eq_v, sel_v, sem):
    wid = lax.axis_index("s") * 2 + lax.axis_index("c")
    rbase = wid * _RW
    pltpu.sync_copy(px_hbm.at[pl.ds(0, _NPAD)], px_v)
    pltpu.sync_copy(py_hbm.at[pl.ds(0, _NPAD)], py_v)
    pltpu.sync_copy(pz_hbm.at[pl.ds(0, _NPAD)], pz_v)
    pltpu.sync_copy(qx_hbm.at[pl.ds(rbase, _RW)], qx_v)
    pltpu.sync_copy(qy_hbm.at[pl.ds(rbase, _RW)], qy_v)
    pltpu.sync_copy(qz_hbm.at[pl.ds(rbase, _RW)], qz_v)
    pltpu.sync_copy(iq_hbm.at[pl.ds(rbase, _RW)], iq_v)

    iota = lax.iota(jnp.int32, 16)
    lane_base = iota * _CAP
    lane_lim = lane_base + (_CAP - 1)

    def group(g, _):
        qxg = qx_v[pl.ds(g * 16, 16)]
        qyg = qy_v[pl.ds(g * 16, 16)]
        qzg = qz_v[pl.ds(g * 16, 16)]

        def scan(c, off):
            lx = px_v[pl.ds(c * 16, 16)]
            ly = py_v[pl.ds(c * 16, 16)]
            lz = pz_v[pl.ds(c * 16, 16)]
            for l in range(16):
                sp = jnp.full((16,), l, jnp.int32)
                dx = jnp.take(lx, sp) - qxg
                dy = jnp.take(ly, sp) - qyg
                dz = jnp.take(lz, sp) - qzg
                d = dx * dx + (dy * dy + dz * dz)
                mm = d < _R2
                tgt = jnp.minimum(off, lane_lim)
                plsc.store_scatter(cand_v, [tgt], c * 16 + l + jnp.zeros((16,), jnp.int32), mask=mm)
                off = off + mm.astype(jnp.int32)
            return off

        off = lax.fori_loop(0, _NPAD // 16, scan, lane_base)

        for l in range(16):
            rloc = g * 16 + l
            cnt = jnp.minimum(off[l] - l * _CAP, _CAP - 1)
            k_take = jnp.minimum(cnt, _K)
            nc = (cnt + 15) // 16
            qxb = jnp.take(qxg, jnp.full((16,), l, jnp.int32))
            qyb = jnp.take(qyg, jnp.full((16,), l, jnp.int32))
            qzb = jnp.take(qzg, jnp.full((16,), l, jnp.int32))

            def rebuild(ch, _):
                ci = cand_v[pl.ds(l * _CAP + ch * 16, 16)]
                ci = jnp.clip(ci, 0, _NPAD - 1)
                gx = plsc.load_gather(px_v, [ci]) - qxb
                gy = plsc.load_gather(py_v, [ci]) - qyb
                gz = plsc.load_gather(pz_v, [ci]) - qzb
                d = gx * gx + (gy * gy + gz * gz)
                db = plsc.bitcast(d, jnp.int32)
                lane = ch * 16 + iota
                db = jnp.where(lane < cnt, db, _BIGBITS)
                bits_v[pl.ds(ch * 16, 16)] = db
                return 0

            lax.fori_loop(0, nc, rebuild, 0)

            def count_le(t):
                def cbody(ch, acc):
                    db = bits_v[pl.ds(ch * 16, 16)]
                    return acc + plsc.all_reduce_population_count(db <= t)

                acc = lax.fori_loop(0, nc, cbody, jnp.zeros((16,), jnp.int32))
                return acc[0]

            def bis(_, lohi):
                lo, hi = lohi
                mid = lo + (hi - lo) // 2
                c = count_le(mid)
                return jnp.where(c >= k_take, lo, mid), jnp.where(c >= k_take, mid, hi)

            lo, hi = lax.fori_loop(0, 31, bis, (jnp.int32(-1), jnp.int32(_R2BITS)))
            t_star = hi
            c_lt = count_le(t_star - 1)
            need_eq = k_take - c_lt

            self_sp = plsc.load_gather(iq_v, [jnp.full((16,), rloc, jnp.int32)])
            for ch4 in range(4):
                sel_v[pl.ds(rloc * _K + ch4 * 16, 16)] = self_sp

            def emit(ch, offs):
                o_lt, o_eq = offs
                db = bits_v[pl.ds(ch * 16, 16)]
                ci = cand_v[pl.ds(l * _CAP + ch * 16, 16)]
                m_lt = db < t_star
                m_eq = db == t_star
                plsc.store_compressed(sel_v.at[pl.ds(rloc * _K + o_lt, 16)], ci, mask=m_lt)
                plsc.store_compressed(eq_v.at[pl.ds(o_eq, 16)], ci, mask=m_eq)
                return (o_lt + plsc.all_reduce_population_count(m_lt)[0],
                        o_eq + plsc.all_reduce_population_count(m_eq)[0])

            lax.fori_loop(0, nc, emit, (jnp.int32(0), jnp.int32(0)))

            for ch4 in range(4):
                mpref = (ch4 * 16 + iota) < need_eq
                ev = eq_v[pl.ds(ch4 * 16, 16)]
                o3 = rloc * _K + jnp.minimum(c_lt + ch4 * 16, _K)
                plsc.store_compressed(sel_v.at[pl.ds(o3, 16)], ev, mask=mpref)
        return 0

    lax.fori_loop(0, _RW // 16, group, 0)
    pltpu.sync_copy(sel_v.at[pl.ds(0, _RW * _K)], nbr_hbm.at[pl.ds(rbase * _K, _RW * _K)])


def _ballq(px, py, pz, qx, qy, qz, iq):
    f = functools.partial(
        pl.kernel,
        out_type=jax.ShapeDtypeStruct((_MPAD * _K,), jnp.int32),
        mesh=_sc_mesh(),
        scratch_types=[
            pltpu.VMEM((_NPAD,), jnp.float32),
            pltpu.VMEM((_NPAD,), jnp.float32),
            pltpu.VMEM((_NPAD,), jnp.float32),
            pltpu.VMEM((_RW,), jnp.float32),
            pltpu.VMEM((_RW,), jnp.float32),
            pltpu.VMEM((_RW,), jnp.float32),
            pltpu.VMEM((_RW,), jnp.int32),
            pltpu.VMEM((16 * _CAP,), jnp.int32),
            pltpu.VMEM((_CAP + 16,), jnp.int32),
            pltpu.VMEM((_CAP + 16,), jnp.int32),
            pltpu.VMEM((_RW * _K + 16,), jnp.int32),
            pltpu.SemaphoreType.DMA,
        ],
        compiler_params=pltpu.CompilerParams(needs_layout_passes=False),
    )(_ballq_body)
    return f(px, py, pz, qx, qy, qz, iq)


_GC = 256
_NCH = _RW * _K // _GC


def _gather_body(x_hbm, nbr_hbm, px_hbm, py_hbm, pz_hbm, qx_hbm, qy_hbm, qz_hbm,
                 xg_hbm, rx_hbm, ry_hbm, rz_hbm,
                 px_v, py_v, pz_v, qx_v, qy_v, qz_v,
                 idx_a, idx_b, rows_a, rows_b,
                 rx_v, ry_v, rz_v, sem_a, sem_b):
    wid = lax.axis_index("s") * 2 + lax.axis_index("c")
    rbase = wid * _RW
    fbase = rbase * _K
    pltpu.sync_copy(px_hbm.at[pl.ds(0, _NPAD)], px_v)
    pltpu.sync_copy(py_hbm.at[pl.ds(0, _NPAD)], py_v)
    pltpu.sync_copy(pz_hbm.at[pl.ds(0, _NPAD)], pz_v)
    pltpu.sync_copy(qx_hbm.at[pl.ds(rbase, _RW)], qx_v)
    pltpu.sync_copy(qy_hbm.at[pl.ds(rbase, _RW)], qy_v)
    pltpu.sync_copy(qz_hbm.at[pl.ds(rbase, _RW)], qz_v)

    def rel_and_out(ch, idx_v, rows_v, sem):
        def sub(sc, _):
            ci = idx_v[pl.ds(sc * 16, 16)]
            rloc = ch * (_GC // _K) + sc // 4
            sp = jnp.full((16,), rloc, jnp.int32)
            gx = plsc.load_gather(px_v, [ci]) - plsc.load_gather(qx_v, [sp])
            gy = plsc.load_gather(py_v, [ci]) - plsc.load_gather(qy_v, [sp])
            gz = plsc.load_gather(pz_v, [ci]) - plsc.load_gather(qz_v, [sp])
            rx_v[pl.ds(sc * 16, 16)] = gx
            ry_v[pl.ds(sc * 16, 16)] = gy
            rz_v[pl.ds(sc * 16, 16)] = gz
            return 0

        lax.fori_loop(0, _GC // 16, sub, 0)
        pltpu.sync_copy(rx_v, rx_hbm.at[pl.ds(fbase + ch * _GC, _GC)])
        pltpu.sync_copy(ry_v, ry_hbm.at[pl.ds(fbase + ch * _GC, _GC)])
        pltpu.sync_copy(rz_v, rz_hbm.at[pl.ds(fbase + ch * _GC, _GC)])
        pltpu.make_async_copy(x_hbm.at[idx_v], rows_v, sem).wait()
        pltpu.sync_copy(rows_v, xg_hbm.at[pl.ds(fbase + ch * _GC, _GC)])

    pltpu.sync_copy(nbr_hbm.at[pl.ds(fbase, _GC)], idx_a)
    pltpu.async_copy(x_hbm.at[idx_a], rows_a, sem_a)

    def pair(i, _):
        chb = 2 * i + 1
        pltpu.sync_copy(nbr_hbm.at[pl.ds(fbase + chb * _GC, _GC)], idx_b)
        pltpu.async_copy(x_hbm.at[idx_b], rows_b, sem_b)
        rel_and_out(2 * i, idx_a, rows_a, sem_a)

        @pl.when(i < _NCH // 2 - 1)
        def _():
            cha = 2 * i + 2
            pltpu.sync_copy(nbr_hbm.at[pl.ds(fbase + cha * _GC, _GC)], idx_a)
            pltpu.async_copy(x_hbm.at[idx_a], rows_a, sem_a)

        rel_and_out(chb, idx_b, rows_b, sem_b)
        return 0

    lax.fori_loop(0, _NCH // 2, pair, 0)


def _gather(x, nbr_flat, px, py, pz, qx, qy, qz):
    f = functools.partial(
        pl.kernel,
        out_type=(
            jax.ShapeDtypeStruct((_MPAD * _K, 128), jnp.float32),
            jax.ShapeDtypeStruct((_MPAD * _K,), jnp.float32),
            jax.ShapeDtypeStruct((_MPAD * _K,), jnp.float32),
            jax.ShapeDtypeStruct((_MPAD * _K,), jnp.float32),
        ),
        mesh=_sc_mesh(),
        scratch_types=[
            pltpu.VMEM((_NPAD,), jnp.float32),
            pltpu.VMEM((_NPAD,), jnp.float32),
            pltpu.VMEM((_NPAD,), jnp.float32),
            pltpu.VMEM((_RW,), jnp.float32),
            pltpu.VMEM((_RW,), jnp.float32),
            pltpu.VMEM((_RW,), jnp.float32),
            pltpu.VMEM((_GC,), jnp.int32),
            pltpu.VMEM((_GC,), jnp.int32),
            pltpu.VMEM((_GC, 128), jnp.float32),
            pltpu.VMEM((_GC, 128), jnp.float32),
            pltpu.VMEM((_GC,), jnp.float32),
            pltpu.VMEM((_GC,), jnp.float32),
            pltpu.VMEM((_GC,), jnp.float32),
            pltpu.SemaphoreType.DMA,
            pltpu.SemaphoreType.DMA,
        ],
        compiler_params=pltpu.CompilerParams(needs_layout_passes=False),
    )(_gather_body)
    return f(x, nbr_flat, px, py, pz, qx, qy, qz)


_CB = 64


def _conv_body(xj_ref, rx_ref, ry_ref, rz_ref, w1x_ref, p0_ref, p1_ref, p2_ref,
               b1_ref, w2_ref, b2_ref, out_ref):
    h = jnp.dot(xj_ref[...], w1x_ref[...], preferred_element_type=jnp.float32)
    h = h + rx_ref[...] * p0_ref[...]
    h = h + ry_ref[...] * p1_ref[...]
    h = h + rz_ref[...] * p2_ref[...]
    h = jnp.maximum(h + b1_ref[...], 0.0)
    h = jnp.dot(h, w2_ref[...], preferred_element_type=jnp.float32) + b2_ref[...]
    h = jnp.maximum(h, 0.0)
    out_ref[...] = jnp.max(h.reshape(_CB, _K, 128), axis=1)


def _conv(xj, rx, ry, rz, W1, b1, W2, b2):
    w1x = W1[:128]
    grid = _MPAD // _CB
    full = lambda i: (0, 0)
    return pl.pallas_call(
        _conv_body,
        grid=(grid,),
        in_specs=[
            pl.BlockSpec((_CB * _K, 128), lambda i: (i, 0)),
            pl.BlockSpec((_CB * _K, 1), lambda i: (i, 0)),
            pl.BlockSpec((_CB * _K, 1), lambda i: (i, 0)),
            pl.BlockSpec((_CB * _K, 1), lambda i: (i, 0)),
            pl.BlockSpec((128, 128), full),
            pl.BlockSpec((1, 128), full),
            pl.BlockSpec((1, 128), full),
            pl.BlockSpec((1, 128), full),
            pl.BlockSpec((1, 128), full),
            pl.BlockSpec((128, 128), full),
            pl.BlockSpec((1, 128), full),
        ],
        out_specs=pl.BlockSpec((_CB, 128), lambda i: (i, 0)),
        out_shape=jax.ShapeDtypeStruct((_MPAD, 128), jnp.float32),
    )(xj, rx.reshape(-1, 1), ry.reshape(-1, 1), rz.reshape(-1, 1),
      w1x, W1[128].reshape(1, 128), W1[129].reshape(1, 128), W1[130].reshape(1, 128),
      b1.reshape(1, 128), W2, b2.reshape(1, 128))


def kernel(x, pos, batch, W1, b1, W2, b2):
    idx = _fps(pos)
    pos_q = pos[idx]

    pad = _NPAD - _N
    px = jnp.pad(pos[:, 0], (0, pad), constant_values=1e3)
    py = jnp.pad(pos[:, 1], (0, pad), constant_values=1e3)
    pz = jnp.pad(pos[:, 2], (0, pad), constant_values=1e3)
    qpad = _MPAD - _M
    qx = jnp.pad(pos_q[:, 0], (0, qpad), constant_values=2e3)
    qy = jnp.pad(pos_q[:, 1], (0, qpad), constant_values=2e3)
    qz = jnp.pad(pos_q[:, 2], (0, qpad), constant_values=2e3)
    iq = jnp.pad(idx, (0, qpad))

    nbr_flat = _ballq(px, py, pz, qx, qy, qz, iq)
    xg, rx, ry, rz = _gather(x, nbr_flat, px, py, pz, qx, qy, qz)
    out = _conv(xg, rx, ry, rz, W1, b1, W2, b2)[:_M]
    return (out, pos_q, batch[idx])


# speedup vs baseline: 17.0715x; 1.0005x over previous
import functools

import jax
import jax.numpy as jnp
from jax import lax
from jax.experimental import pallas as pl
from jax.experimental.pallas import tpu as pltpu
from jax.experimental.pallas import tpu_sc as plsc

_N = 10000
_M = 2500
_R2 = 0.2 * 0.2
_K = 64
_NPAD = 10240
_ROWS = 80
_MPAD = 2560
_NW = 32
_RW = _MPAD // _NW
_CAP = 768
_BIGBITS = 0x7F700000
_R2BITS = __import__("struct").unpack("<i", __import__("struct").pack("<f", _R2))[0]


def _sc_mesh():
    return plsc.VectorSubcoreMesh(
        core_axis_name="c", subcore_axis_name="s", num_cores=2, num_subcores=16)


def _fps_body(px_ref, py_ref, pz_ref, pxs_ref, pys_ref, pzs_ref, mind0_ref,
              out_ref):
    flat = (lax.broadcasted_iota(jnp.int32, (_ROWS, 128), 0) * 128
            + lax.broadcasted_iota(jnp.int32, (_ROWS, 128), 1))
    out_ref[0] = 0
    px = px_ref[...]
    py = py_ref[...]
    pz = pz_ref[...]

    def body(i, mind):
        m = jnp.max(mind)
        eq = mind == m
        nxt = jnp.min(jnp.where(eq, flat, jnp.int32(2**30)))
        qx = pxs_ref[nxt]
        qy = pys_ref[nxt]
        qz = pzs_ref[nxt]
        dx = px - qx
        dy = py - qy
        dz = pz - qz
        d = dx * dx + (dy * dy + dz * dz)
        out_ref[i] = nxt
        return jnp.minimum(mind, d)

    lax.fori_loop(1, _M, body, mind0_ref[...])


def _fps(pos):
    pad = _NPAD - _N
    pxf = jnp.pad(pos[:, 0], (0, pad))
    pyf = jnp.pad(pos[:, 1], (0, pad))
    pzf = jnp.pad(pos[:, 2], (0, pad))
    px = pxf.reshape(_ROWS, 128)
    py = pyf.reshape(_ROWS, 128)
    pz = pzf.reshape(_ROWS, 128)
    d0 = jnp.sum((pos - pos[0]) ** 2, axis=1)
    mind0 = jnp.pad(d0, (0, pad), constant_values=-1.0).reshape(_ROWS, 128)
    return pl.pallas_call(
        _fps_body,
        out_shape=jax.ShapeDtypeStruct((_M,), jnp.int32),
        in_specs=[pl.BlockSpec(memory_space=pltpu.VMEM)] * 3
        + [pl.BlockSpec(memory_space=pltpu.SMEM)] * 3
        + [pl.BlockSpec(memory_space=pltpu.VMEM)],
        out_specs=pl.BlockSpec(memory_space=pltpu.SMEM),
    )(px, py, pz, pxf, pyf, pzf, mind0)


def _ballq_body(px_hbm, py_hbm, pz_hbm, qx_hbm, qy_hbm, qz_hbm, iq_hbm,
                nbr_hbm, px_v, py_v, pz_v, qx_v, qy_v, qz_v, iq_v,
                cand_v, bits_v, eq_v, sel_v, sem):
    wid = lax.axis_index("s") * 2 + lax.axis_index("c")
    rbase = wid * _RW
    pltpu.sync_copy(px_hbm.at[pl.ds(0, _NPAD)], px_v)
    pltpu.sync_copy(py_hbm.at[pl.ds(0, _NPAD)], py_v)
    pltpu.sync_copy(pz_hbm.at[pl.ds(0, _NPAD)], pz_v)
    pltpu.sync_copy(qx_hbm.at[pl.ds(rbase, _RW)], qx_v)
    pltpu.sync_copy(qy_hbm.at[pl.ds(rbase, _RW)], qy_v)
    pltpu.sync_copy(qz_hbm.at[pl.ds(rbase, _RW)], qz_v)
    pltpu.sync_copy(iq_hbm.at[pl.ds(rbase, _RW)], iq_v)

    iota = lax.iota(jnp.int32, 16)
    lane_base = iota * _CAP
    lane_lim = lane_base + (_CAP - 1)

    def group(g, _):
        qxg = qx_v[pl.ds(g * 16, 16)]
        qyg = qy_v[pl.ds(g * 16, 16)]
        qzg = qz_v[pl.ds(g * 16, 16)]

        def scan(c, off):
            lx = px_v[pl.ds(c * 16, 16)]
            ly = py_v[pl.ds(c * 16, 16)]
            lz = pz_v[pl.ds(c * 16, 16)]
            for l in range(16):
                sp = jnp.full((16,), l, jnp.int32)
                dx = jnp.take(lx, sp) - qxg
                dy = jnp.take(ly, sp) - qyg
                dz = jnp.take(lz, sp) - qzg
                d = dx * dx + (dy * dy + dz * dz)
                mm = d < _R2
                tgt = jnp.minimum(off, lane_lim)
                plsc.store_scatter(cand_v, [tgt], c * 16 + l + jnp.zeros((16,), jnp.int32), mask=mm)
                off = off + mm.astype(jnp.int32)
            return off

        off = lax.fori_loop(0, _NPAD // 16, scan, lane_base)

        for l in range(16):
            rloc = g * 16 + l
            cnt = jnp.minimum(off[l] - l * _CAP, _CAP - 1)
            k_take = jnp.minimum(cnt, _K)
            nc = (cnt + 15) // 16
            qxb = jnp.take(qxg, jnp.full((16,), l, jnp.int32))
            qyb = jnp.take(qyg, jnp.full((16,), l, jnp.int32))
            qzb = jnp.take(qzg, jnp.full((16,), l, jnp.int32))

            def rebuild(ch, _):
                ci = cand_v[pl.ds(l * _CAP + ch * 16, 16)]
                ci = jnp.clip(ci, 0, _NPAD - 1)
                gx = plsc.load_gather(px_v, [ci]) - qxb
                gy = plsc.load_gather(py_v, [ci]) - qyb
                gz = plsc.load_gather(pz_v, [ci]) - qzb
                d = gx * gx + (gy * gy + gz * gz)
                db = plsc.bitcast(d, jnp.int32)
                lane = ch * 16 + iota
                db = jnp.where(lane < cnt, db, _BIGBITS)
                bits_v[pl.ds(ch * 16, 16)] = db
                return 0

            lax.fori_loop(0, nc, rebuild, 0)

            def count_le(t):
                def cbody(ch, acc):
                    db = bits_v[pl.ds(ch * 16, 16)]
                    return acc + plsc.all_reduce_population_count(db <= t)

                acc = lax.fori_loop(0, nc, cbody, jnp.zeros((16,), jnp.int32))
                return acc[0]

            def bis(_, lohi):
                lo, hi = lohi
                mid = lo + (hi - lo) // 2
                c = count_le(mid)
                return jnp.where(c >= k_take, lo, mid), jnp.where(c >= k_take, mid, hi)

            lo, hi = lax.fori_loop(0, 31, bis, (jnp.int32(-1), jnp.int32(_R2BITS)))
            t_star = hi
            c_lt = count_le(t_star - 1)
            need_eq = k_take - c_lt

            self_sp = plsc.load_gather(iq_v, [jnp.full((16,), rloc, jnp.int32)])
            for ch4 in range(4):
                sel_v[pl.ds(rloc * _K + ch4 * 16, 16)] = self_sp

            def emit(ch, offs):
                o_lt, o_eq = offs
                db = bits_v[pl.ds(ch * 16, 16)]
                ci = cand_v[pl.ds(l * _CAP + ch * 16, 16)]
                m_lt = db < t_star
                m_eq = db == t_star
                plsc.store_compressed(sel_v.at[pl.ds(rloc * _K + o_lt, 16)], ci, mask=m_lt)
                plsc.store_compressed(eq_v.at[pl.ds(o_eq, 16)], ci, mask=m_eq)
                return (o_lt + plsc.all_reduce_population_count(m_lt)[0],
                        o_eq + plsc.all_reduce_population_count(m_eq)[0])

            lax.fori_loop(0, nc, emit, (jnp.int32(0), jnp.int32(0)))

            for ch4 in range(4):
                mpref = (ch4 * 16 + iota) < need_eq
                ev = eq_v[pl.ds(ch4 * 16, 16)]
                o3 = rloc * _K + jnp.minimum(c_lt + ch4 * 16, _K)
                plsc.store_compressed(sel_v.at[pl.ds(o3, 16)], ev, mask=mpref)
        return 0

    lax.fori_loop(0, _RW // 16, group, 0)
    pltpu.sync_copy(sel_v.at[pl.ds(0, _RW * _K)], nbr_hbm.at[pl.ds(rbase * _K, _RW * _K)])


def _ballq(px, py, pz, qx, qy, qz, iq):
    f = functools.partial(
        pl.kernel,
        out_type=jax.ShapeDtypeStruct((_MPAD * _K,), jnp.int32),
        mesh=_sc_mesh(),
        scratch_types=[
            pltpu.VMEM((_NPAD,), jnp.float32),
            pltpu.VMEM((_NPAD,), jnp.float32),
            pltpu.VMEM((_NPAD,), jnp.float32),
            pltpu.VMEM((_RW,), jnp.float32),
            pltpu.VMEM((_RW,), jnp.float32),
            pltpu.VMEM((_RW,), jnp.float32),
            pltpu.VMEM((_RW,), jnp.int32),
            pltpu.VMEM((16 * _CAP,), jnp.int32),
            pltpu.VMEM((_CAP + 16,), jnp.int32),
            pltpu.VMEM((_CAP + 16,), jnp.int32),
            pltpu.VMEM((_RW * _K + 16,), jnp.int32),
            pltpu.SemaphoreType.DMA,
        ],
        compiler_params=pltpu.CompilerParams(needs_layout_passes=False),
    )(_ballq_body)
    return f(px, py, pz, qx, qy, qz, iq)


_GC = 256
_NCH = _RW * _K // _GC


def _gather_body(x_hbm, nbr_hbm, px_hbm, py_hbm, pz_hbm, qx_hbm, qy_hbm, qz_hbm,
                 xg_hbm, rx_hbm, ry_hbm, rz_hbm,
                 px_v, py_v, pz_v, qx_v, qy_v, qz_v,
                 idx_a, idx_b, rows_a, rows_b,
                 rx_v, ry_v, rz_v, sem_a, sem_b):
    wid = lax.axis_index("s") * 2 + lax.axis_index("c")
    rbase = wid * _RW
    fbase = rbase * _K
    pltpu.sync_copy(px_hbm.at[pl.ds(0, _NPAD)], px_v)
    pltpu.sync_copy(py_hbm.at[pl.ds(0, _NPAD)], py_v)
    pltpu.sync_copy(pz_hbm.at[pl.ds(0, _NPAD)], pz_v)
    pltpu.sync_copy(qx_hbm.at[pl.ds(rbase, _RW)], qx_v)
    pltpu.sync_copy(qy_hbm.at[pl.ds(rbase, _RW)], qy_v)
    pltpu.sync_copy(qz_hbm.at[pl.ds(rbase, _RW)], qz_v)

    def rel_and_out(ch, idx_v, rows_v, sem):
        def sub(sc, _):
            ci = idx_v[pl.ds(sc * 16, 16)]
            rloc = ch * (_GC // _K) + sc // 4
            sp = jnp.full((16,), rloc, jnp.int32)
            gx = plsc.load_gather(px_v, [ci]) - plsc.load_gather(qx_v, [sp])
            gy = plsc.load_gather(py_v, [ci]) - plsc.load_gather(qy_v, [sp])
            gz = plsc.load_gather(pz_v, [ci]) - plsc.load_gather(qz_v, [sp])
            rx_v[pl.ds(sc * 16, 16)] = gx
            ry_v[pl.ds(sc * 16, 16)] = gy
            rz_v[pl.ds(sc * 16, 16)] = gz
            return 0

        lax.fori_loop(0, _GC // 16, sub, 0)
        pltpu.sync_copy(rx_v, rx_hbm.at[pl.ds(fbase + ch * _GC, _GC)])
        pltpu.sync_copy(ry_v, ry_hbm.at[pl.ds(fbase + ch * _GC, _GC)])
        pltpu.sync_copy(rz_v, rz_hbm.at[pl.ds(fbase + ch * _GC, _GC)])
        pltpu.make_async_copy(x_hbm.at[idx_v], rows_v, sem).wait()
        pltpu.sync_copy(rows_v, xg_hbm.at[pl.ds(fbase + ch * _GC, _GC)])

    pltpu.sync_copy(nbr_hbm.at[pl.ds(fbase, _GC)], idx_a)
    pltpu.async_copy(x_hbm.at[idx_a], rows_a, sem_a)

    def pair(i, _):
        chb = 2 * i + 1
        pltpu.sync_copy(nbr_hbm.at[pl.ds(fbase + chb * _GC, _GC)], idx_b)
        pltpu.async_copy(x_hbm.at[idx_b], rows_b, sem_b)
        rel_and_out(2 * i, idx_a, rows_a, sem_a)

        @pl.when(i < _NCH // 2 - 1)
        def _():
            cha = 2 * i + 2
            pltpu.sync_copy(nbr_hbm.at[pl.ds(fbase + cha * _GC, _GC)], idx_a)
            pltpu.async_copy(x_hbm.at[idx_a], rows_a, sem_a)

        rel_and_out(chb, idx_b, rows_b, sem_b)
        return 0

    lax.fori_loop(0, _NCH // 2, pair, 0)


def _gather(x, nbr_flat, px, py, pz, qx, qy, qz):
    f = functools.partial(
        pl.kernel,
        out_type=(
            jax.ShapeDtypeStruct((_MPAD * _K, 128), jnp.float32),
            jax.ShapeDtypeStruct((_MPAD * _K,), jnp.float32),
            jax.ShapeDtypeStruct((_MPAD * _K,), jnp.float32),
            jax.ShapeDtypeStruct((_MPAD * _K,), jnp.float32),
        ),
        mesh=_sc_mesh(),
        scratch_types=[
            pltpu.VMEM((_NPAD,), jnp.float32),
            pltpu.VMEM((_NPAD,), jnp.float32),
            pltpu.VMEM((_NPAD,), jnp.float32),
            pltpu.VMEM((_RW,), jnp.float32),
            pltpu.VMEM((_RW,), jnp.float32),
            pltpu.VMEM((_RW,), jnp.float32),
            pltpu.VMEM((_GC,), jnp.int32),
            pltpu.VMEM((_GC,), jnp.int32),
            pltpu.VMEM((_GC, 128), jnp.float32),
            pltpu.VMEM((_GC, 128), jnp.float32),
            pltpu.VMEM((_GC,), jnp.float32),
            pltpu.VMEM((_GC,), jnp.float32),
            pltpu.VMEM((_GC,), jnp.float32),
            pltpu.SemaphoreType.DMA,
            pltpu.SemaphoreType.DMA,
        ],
        compiler_params=pltpu.CompilerParams(needs_layout_passes=False),
    )(_gather_body)
    return f(x, nbr_flat, px, py, pz, qx, qy, qz)


_CB = 64


def _conv_body(xj_ref, rx_ref, ry_ref, rz_ref, w1x_ref, p0_ref, p1_ref, p2_ref,
               b1_ref, w2_ref, b2_ref, out_ref):
    h = jnp.dot(xj_ref[...], w1x_ref[...], preferred_element_type=jnp.float32)
    h = h + rx_ref[...] * p0_ref[...]
    h = h + ry_ref[...] * p1_ref[...]
    h = h + rz_ref[...] * p2_ref[...]
    h = jnp.maximum(h + b1_ref[...], 0.0)
    h = jnp.dot(h, w2_ref[...], preferred_element_type=jnp.float32) + b2_ref[...]
    h = jnp.maximum(h, 0.0)
    out_ref[...] = jnp.max(h.reshape(_CB, _K, 128), axis=1)


def _conv(xj, rx, ry, rz, W1, b1, W2, b2):
    w1x = W1[:128]
    grid = _MPAD // _CB
    full = lambda i: (0, 0)
    return pl.pallas_call(
        _conv_body,
        grid=(grid,),
        in_specs=[
            pl.BlockSpec((_CB * _K, 128), lambda i: (i, 0)),
            pl.BlockSpec((_CB * _K, 1), lambda i: (i, 0)),
            pl.BlockSpec((_CB * _K, 1), lambda i: (i, 0)),
            pl.BlockSpec((_CB * _K, 1), lambda i: (i, 0)),
            pl.BlockSpec((128, 128), full),
            pl.BlockSpec((1, 128), full),
            pl.BlockSpec((1, 128), full),
            pl.BlockSpec((1, 128), full),
            pl.BlockSpec((1, 128), full),
            pl.BlockSpec((128, 128), full),
            pl.BlockSpec((1, 128), full),
        ],
        out_specs=pl.BlockSpec((_CB, 128), lambda i: (i, 0)),
        out_shape=jax.ShapeDtypeStruct((_MPAD, 128), jnp.float32),
    )(xj, rx.reshape(-1, 1), ry.reshape(-1, 1), rz.reshape(-1, 1),
      w1x, W1[128].reshape(1, 128), W1[129].reshape(1, 128), W1[130].reshape(1, 128),
      b1.reshape(1, 128), W2, b2.reshape(1, 128))


def kernel(x, pos, batch, W1, b1, W2, b2):
    idx = _fps(pos)
    pos_q = pos[idx]

    pad = _NPAD - _N
    px = jnp.pad(pos[:, 0], (0, pad), constant_values=1e3)
    py = jnp.pad(pos[:, 1], (0, pad), constant_values=1e3)
    pz = jnp.pad(pos[:, 2], (0, pad), constant_values=1e3)
    qpad = _MPAD - _M
    qx = jnp.pad(pos_q[:, 0], (0, qpad), constant_values=2e3)
    qy = jnp.pad(pos_q[:, 1], (0, qpad), constant_values=2e3)
    qz = jnp.pad(pos_q[:, 2], (0, qpad), constant_values=2e3)
    iq = jnp.pad(idx, (0, qpad))

    nbr_flat = _ballq(px, py, pz, qx, qy, qz, iq)
    xg, rx, ry, rz = _gather(x, nbr_flat, px, py, pz, qx, qy, qz)
    out = _conv(xg, rx, ry, rz, W1, b1, W2, b2)[:_M]
    return (out, pos_q, batch[idx])
